# baseline (speedup 1.0000x reference)
import functools

import jax
import jax.numpy as jnp
from jax import lax
from jax.experimental import pallas as pl
from jax.experimental.pallas import tpu as pltpu
from jax.experimental.pallas import tpu_sc as plsc

B = 16384
D = 128
K = 20
NC = 2
NS = 16
NW = NC * NS
BPW = B // NW
C = 8
NCHUNK = BPW // C
CK = C * K
L = 16
DV = D // L
NEG_SLICES = [(o, min(128, CK - o)) for o in range(0, CK, 128)]


def _sc_dots(iw_hbm, cw_hbm, neg_hbm, win_hbm, wctx_hbm,
             pos_hbm, nout_hbm,
             iwcw_a, iwcw_b, nidx_a, nidx_b, inctx_a, inctx_b,
             neg_a, neg_b, pos_buf, neg_buf,
             sem_idx_a, sem_idx_b, sem_g_a, sem_g_b, sem_out):
    wid = lax.axis_index("s") * NC + lax.axis_index("c")

    bufs = [
        (iwcw_a, nidx_a, inctx_a, neg_a, sem_idx_a, sem_g_a),
        (iwcw_b, nidx_b, inctx_b, neg_b, sem_idx_b, sem_g_b),
    ]

    def stage_idx(cc, p):
        iwcw_v, nidx_v, _, _, sem, _ = bufs[p]
        b0 = wid * BPW + cc * C
        pltpu.async_copy(iw_hbm.at[pl.ds(b0, C)], iwcw_v.at[pl.ds(0, C)], sem)
        pltpu.async_copy(cw_hbm.at[pl.ds(b0, C)], iwcw_v.at[pl.ds(C, C)], sem)
        pltpu.async_copy(neg_hbm.at[pl.ds(b0 * K, CK)], nidx_v, sem)

    def wait_idx(p):
        iwcw_v, nidx_v, _, _, sem, _ = bufs[p]
        pltpu.make_async_copy(iw_hbm.at[pl.ds(0, 2 * C)], iwcw_v, sem).wait()
        pltpu.make_async_copy(neg_hbm.at[pl.ds(0, CK)], nidx_v, sem).wait()

    def start_gathers(p):
        iwcw_v, nidx_v, inctx_rows, neg_rows, _, sem = bufs[p]
        pltpu.async_copy(win_hbm.at[iwcw_v.at[pl.ds(0, C)]],
                         inctx_rows.at[pl.ds(0, C)], sem)
        pltpu.async_copy(wctx_hbm.at[iwcw_v.at[pl.ds(C, C)]],
                         inctx_rows.at[pl.ds(C, C)], sem)
        for off, sz in NEG_SLICES:
            pltpu.async_copy(wctx_hbm.at[nidx_v.at[pl.ds(off, sz)]],
                             neg_rows.at[pl.ds(off, sz)], sem)

    def wait_gathers(p):
        iwcw_v, nidx_v, inctx_rows, neg_rows, _, sem = bufs[p]
        pltpu.make_async_copy(win_hbm.at[iwcw_v], inctx_rows, sem).wait()
        pltpu.make_async_copy(wctx_hbm.at[nidx_v], neg_rows, sem).wait()

    def out_copies(cc, make_only):
        f = pltpu.make_async_copy if make_only else pltpu.async_copy
        base = wid * NCHUNK + cc
        return [
            f(pos_buf, pos_hbm.at[pl.ds(base * (C * L), C * L)], sem_out),
            f(neg_buf, nout_hbm.at[pl.ds(base * (CK * L), CK * L)], sem_out),
        ]

    def issue_out(cc):
        out_copies(cc, make_only=False)

    def wait_out(cc):
        for cp in out_copies(cc, make_only=True):
            cp.wait()

    def compute(p):
        _, _, inctx_rows, neg_rows, _, _ = bufs[p]

        @plsc.parallel_loop(0, C, unroll=2)
        def b_body(b):
            iv = [inctx_rows[b, pl.ds(L * d, L)] for d in range(DV)]

            def dot_row(rows, r):
                prod = [iv[d] * rows[r, pl.ds(L * d, L)] for d in range(DV)]
                return ((prod[0] + prod[1]) + (prod[2] + prod[3])) + \
                       ((prod[4] + prod[5]) + (prod[6] + prod[7]))

            pos_buf[pl.ds(b * L, L)] = dot_row(inctx_rows, C + b)
            for k in range(K):
                r = b * K + k
                neg_buf[pl.ds(r * L, L)] = dot_row(neg_rows, r)

    stage_idx(0, 0)
    stage_idx(1, 1)
    wait_idx(0)
    start_gathers(0)

    def pair_body(t, carry):
        cc0 = 2 * t
        cc1 = 2 * t + 1

        wait_gathers(0)

        @pl.when(cc0 + 2 < NCHUNK)
        def _():
            stage_idx(cc0 + 2, 0)

        wait_idx(1)
        start_gathers(1)

        @pl.when(t >= 1)
        def _():
            wait_out(cc0 - 1)

        compute(0)
        issue_out(cc0)

        wait_gathers(1)

        @pl.when(cc1 + 2 < NCHUNK)
        def _():
            stage_idx(cc1 + 2, 1)

        @pl.when(cc0 + 2 < NCHUNK)
        def _():
            wait_idx(0)
            start_gathers(0)

        wait_out(cc0)
        compute(1)
        issue_out(cc1)
        return carry

    lax.fori_loop(0, NCHUNK // 2, pair_body, 0)

    wait_out(NCHUNK - 1)


_sc_dots_call = functools.partial(
    pl.kernel,
    out_type=[
        jax.ShapeDtypeStruct((B * L,), jnp.float32),
        jax.ShapeDtypeStruct((B * K * L,), jnp.float32),
    ],
    mesh=plsc.VectorSubcoreMesh(core_axis_name="c", subcore_axis_name="s"),
    scratch_types=[
        pltpu.VMEM((2 * C,), jnp.int32),
        pltpu.VMEM((2 * C,), jnp.int32),
        pltpu.VMEM((CK,), jnp.int32),
        pltpu.VMEM((CK,), jnp.int32),
        pltpu.VMEM((2 * C, D), jnp.float32),
        pltpu.VMEM((2 * C, D), jnp.float32),
        pltpu.VMEM((CK, D), jnp.float32),
        pltpu.VMEM((CK, D), jnp.float32),
        pltpu.VMEM((C * L,), jnp.float32),
        pltpu.VMEM((CK * L,), jnp.float32),
        pltpu.SemaphoreType.DMA,
        pltpu.SemaphoreType.DMA,
        pltpu.SemaphoreType.DMA,
        pltpu.SemaphoreType.DMA,
        pltpu.SemaphoreType.DMA,
    ],
)(_sc_dots)


def _log_sigmoid(x):
    return jnp.minimum(x, 0.0) - jnp.log1p(jnp.exp(-jnp.abs(x)))


_TCGRID = 4
_PROWS = B * L // 128
_NROWS = B * K * L // 128
_PB = _PROWS // _TCGRID
_NB = _NROWS // _TCGRID


def _tc_loss_body(pos_ref, neg_ref, out_ref):
    i = pl.program_id(0)
    cg = lax.broadcasted_iota(jnp.int32, (128, 8), 0) // L
    g = lax.broadcasted_iota(jnp.int32, (128, 8), 1)
    m = (cg == g).astype(jnp.float32)
    ps = jax.lax.dot(pos_ref[...], m,
                     precision=jax.lax.Precision.DEFAULT,
                     preferred_element_type=jnp.float32)
    ns = jax.lax.dot(neg_ref[...], m,
                     precision=jax.lax.Precision.DEFAULT,
                     preferred_element_type=jnp.float32)
    t = jnp.sum(_log_sigmoid(ps)) + jnp.sum(_log_sigmoid(-ns))

    @pl.when(i == 0)
    def _():
        out_ref[0, 0] = 0.0

    out_ref[0, 0] += -t / B


_tc_loss = pl.pallas_call(
    _tc_loss_body,
    grid=(_TCGRID,),
    in_specs=[
        pl.BlockSpec((_PB, 128), lambda i: (i, 0)),
        pl.BlockSpec((_NB, 128), lambda i: (i, 0)),
    ],
    out_specs=pl.BlockSpec(memory_space=pltpu.MemorySpace.SMEM),
    out_shape=jax.ShapeDtypeStruct((1, 1), jnp.float32),
)


@jax.jit
def kernel(input_word, context_word, W_in, W_ctx, negative_example):
    iw = input_word.astype(jnp.int32)
    cw = context_word.astype(jnp.int32)
    negflat = negative_example.astype(jnp.int32).reshape(B * K)

    pos, neg = _sc_dots_call(iw, cw, negflat, W_in, W_ctx)
    loss = _tc_loss(pos.reshape(_PROWS, 128), neg.reshape(_NROWS, 128))
    return loss[0, 0]


# speedup vs baseline: 1.0373x; 1.0373x over previous
import functools

import jax
import jax.numpy as jnp
from jax import lax
from jax.experimental import pallas as pl
from jax.experimental.pallas import tpu as pltpu
from jax.experimental.pallas import tpu_sc as plsc

B = 16384
D = 128
K = 20
NC = 2
NS = 16
NW = NC * NS
BPW = B // NW
C = 16
NCHUNK = BPW // C
CK = C * K
L = 16
DV = D // L
NEG_SLICES = [(o, min(128, CK - o)) for o in range(0, CK, 128)]


def _sc_dots(iw_hbm, cw_hbm, neg_hbm, win_hbm, wctx_hbm,
             pos_hbm, nout_hbm,
             iwcw_a, iwcw_b, nidx_a, nidx_b, inctx_rows, neg_a, neg_b,
             pos_buf, neg_buf,
             sem_idx_a, sem_idx_b, sem_g_a, sem_g_b, sem_ic, sem_out):
    wid = lax.axis_index("s") * NC + lax.axis_index("c")

    bufs = [
        (iwcw_a, nidx_a, neg_a, sem_idx_a, sem_g_a),
        (iwcw_b, nidx_b, neg_b, sem_idx_b, sem_g_b),
    ]

    def stage_idx(cc, p):
        iwcw_v, nidx_v, _, sem, _ = bufs[p]
        b0 = wid * BPW + cc * C
        pltpu.async_copy(iw_hbm.at[pl.ds(b0, C)], iwcw_v.at[pl.ds(0, C)], sem)
        pltpu.async_copy(cw_hbm.at[pl.ds(b0, C)], iwcw_v.at[pl.ds(C, C)], sem)
        pltpu.async_copy(neg_hbm.at[pl.ds(b0 * K, CK)], nidx_v, sem)

    def wait_idx(p):
        iwcw_v, nidx_v, _, sem, _ = bufs[p]
        pltpu.make_async_copy(iw_hbm.at[pl.ds(0, 2 * C)], iwcw_v, sem).wait()
        pltpu.make_async_copy(neg_hbm.at[pl.ds(0, CK)], nidx_v, sem).wait()

    def start_gathers(p):
        _, nidx_v, neg_rows, _, sem = bufs[p]
        for off, sz in NEG_SLICES:
            pltpu.async_copy(wctx_hbm.at[nidx_v.at[pl.ds(off, sz)]],
                             neg_rows.at[pl.ds(off, sz)], sem)

    def wait_gathers(p):
        _, nidx_v, neg_rows, _, sem = bufs[p]
        pltpu.make_async_copy(wctx_hbm.at[nidx_v], neg_rows, sem).wait()

    def start_ic(p):
        iwcw_v, _, _, _, _ = bufs[p]
        pltpu.async_copy(win_hbm.at[iwcw_v.at[pl.ds(0, C)]],
                         inctx_rows.at[pl.ds(0, C)], sem_ic)
        pltpu.async_copy(wctx_hbm.at[iwcw_v.at[pl.ds(C, C)]],
                         inctx_rows.at[pl.ds(C, C)], sem_ic)

    def wait_ic():
        pltpu.make_async_copy(win_hbm.at[iwcw_a], inctx_rows, sem_ic).wait()

    def out_copies(cc, make_only):
        f = pltpu.make_async_copy if make_only else pltpu.async_copy
        base = wid * NCHUNK + cc
        return [
            f(pos_buf, pos_hbm.at[pl.ds(base * (C * L), C * L)], sem_out),
            f(neg_buf, nout_hbm.at[pl.ds(base * (CK * L), CK * L)], sem_out),
        ]

    def issue_out(cc):
        out_copies(cc, make_only=False)

    def wait_out(cc):
        for cp in out_copies(cc, make_only=True):
            cp.wait()

    def compute(p):
        _, _, neg_rows, _, _ = bufs[p]

        @plsc.parallel_loop(0, C, unroll=2)
        def b_body(b):
            iv = [inctx_rows[b, pl.ds(L * d, L)] for d in range(DV)]

            def dot_row(rows, r):
                prod = [iv[d] * rows[r, pl.ds(L * d, L)] for d in range(DV)]
                return ((prod[0] + prod[1]) + (prod[2] + prod[3])) + \
                       ((prod[4] + prod[5]) + (prod[6] + prod[7]))

            pos_buf[pl.ds(b * L, L)] = dot_row(inctx_rows, C + b)
            for k in range(K):
                r = b * K + k
                neg_buf[pl.ds(r * L, L)] = dot_row(neg_rows, r)

    stage_idx(0, 0)
    stage_idx(1, 1)
    wait_idx(0)
    start_gathers(0)
    start_ic(0)

    def pair_body(t, carry):
        cc0 = 2 * t
        cc1 = 2 * t + 1

        wait_gathers(0)

        @pl.when(cc0 + 2 < NCHUNK)
        def _():
            stage_idx(cc0 + 2, 0)

        wait_idx(1)
        start_gathers(1)

        @pl.when(t >= 1)
        def _():
            wait_out(cc0 - 1)

        wait_ic()
        compute(0)
        start_ic(1)
        issue_out(cc0)

        wait_gathers(1)

        @pl.when(cc1 + 2 < NCHUNK)
        def _():
            stage_idx(cc1 + 2, 1)

        @pl.when(cc0 + 2 < NCHUNK)
        def _():
            wait_idx(0)
            start_gathers(0)

        wait_out(cc0)
        wait_ic()
        compute(1)

        @pl.when(cc0 + 2 < NCHUNK)
        def _():
            start_ic(0)

        issue_out(cc1)
        return carry

    lax.fori_loop(0, NCHUNK // 2, pair_body, 0)

    wait_out(NCHUNK - 1)


_sc_dots_call = functools.partial(
    pl.kernel,
    out_type=[
        jax.ShapeDtypeStruct((B * L,), jnp.float32),
        jax.ShapeDtypeStruct((B * K * L,), jnp.float32),
    ],
    mesh=plsc.VectorSubcoreMesh(core_axis_name="c", subcore_axis_name="s"),
    scratch_types=[
        pltpu.VMEM((2 * C,), jnp.int32),
        pltpu.VMEM((2 * C,), jnp.int32),
        pltpu.VMEM((CK,), jnp.int32),
        pltpu.VMEM((CK,), jnp.int32),
        pltpu.VMEM((2 * C, D), jnp.float32),
        pltpu.VMEM((CK, D), jnp.float32),
        pltpu.VMEM((CK, D), jnp.float32),
        pltpu.VMEM((C * L,), jnp.float32),
        pltpu.VMEM((CK * L,), jnp.float32),
        pltpu.SemaphoreType.DMA,
        pltpu.SemaphoreType.DMA,
        pltpu.SemaphoreType.DMA,
        pltpu.SemaphoreType.DMA,
        pltpu.SemaphoreType.DMA,
        pltpu.SemaphoreType.DMA,
    ],
)(_sc_dots)


def _log_sigmoid(x):
    return jnp.minimum(x, 0.0) - jnp.log1p(jnp.exp(-jnp.abs(x)))


_TCGRID = 4
_PROWS = B * L // 128
_NROWS = B * K * L // 128
_PB = _PROWS // _TCGRID
_NB = _NROWS // _TCGRID


def _tc_loss_body(pos_ref, neg_ref, out_ref):
    i = pl.program_id(0)
    cg = lax.broadcasted_iota(jnp.int32, (128, 8), 0) // L
    g = lax.broadcasted_iota(jnp.int32, (128, 8), 1)
    m = (cg == g).astype(jnp.float32)
    ps = jax.lax.dot(pos_ref[...], m,
                     precision=jax.lax.Precision.DEFAULT,
                     preferred_element_type=jnp.float32)
    ns = jax.lax.dot(neg_ref[...], m,
                     precision=jax.lax.Precision.DEFAULT,
                     preferred_element_type=jnp.float32)
    t = jnp.sum(_log_sigmoid(ps)) + jnp.sum(_log_sigmoid(-ns))

    @pl.when(i == 0)
    def _():
        out_ref[0, 0] = 0.0

    out_ref[0, 0] += -t / B


_tc_loss = pl.pallas_call(
    _tc_loss_body,
    grid=(_TCGRID,),
    in_specs=[
        pl.BlockSpec((_PB, 128), lambda i: (i, 0)),
        pl.BlockSpec((_NB, 128), lambda i: (i, 0)),
    ],
    out_specs=pl.BlockSpec(memory_space=pltpu.MemorySpace.SMEM),
    out_shape=jax.ShapeDtypeStruct((1, 1), jnp.float32),
)


@jax.jit
def kernel(input_word, context_word, W_in, W_ctx, negative_example):
    iw = input_word.astype(jnp.int32)
    cw = context_word.astype(jnp.int32)
    negflat = negative_example.astype(jnp.int32).reshape(B * K)

    pos, neg = _sc_dots_call(iw, cw, negflat, W_in, W_ctx)
    loss = _tc_loss(pos.reshape(_PROWS, 128), neg.reshape(_NROWS, 128))
    return loss[0, 0]
